# baseline (speedup 1.0000x reference)
import functools

import jax
import jax.numpy as jnp
from jax import lax
from jax.experimental import pallas as pl
from jax.experimental.pallas import tpu as pltpu
from jax.experimental.pallas import tpu_sc as plsc

_NUM_CORES = 2
_NUM_SUBCORES = 16
_NUM_WORKERS = _NUM_CORES * _NUM_SUBCORES

_CB = 256
_L = 16


@functools.lru_cache(maxsize=None)
def _make_gather(batch: int, hist: int, vocab: int, dim: int):
    assert dim == 64 and vocab % 2 == 0 and batch % _CB == 0
    n_units = hist * (batch // _CB)
    assert n_units % _NUM_WORKERS == 0
    units_per_w = n_units // _NUM_WORKERS
    chunks_per_h = batch // _CB
    mesh = plsc.VectorSubcoreMesh(
        core_axis_name="c", subcore_axis_name="s",
        num_cores=_NUM_CORES, num_subcores=_NUM_SUBCORES)

    @functools.partial(
        pl.kernel,
        mesh=mesh,
        out_type=jax.ShapeDtypeStruct((hist, dim, batch), jnp.float32),
        scratch_types=[
            pltpu.VMEM((_CB,), jnp.int32),
            pltpu.VMEM((_CB,), jnp.int32),
            pltpu.VMEM((_CB, 2 * dim), jnp.float32),
            pltpu.VMEM((dim, _CB), jnp.float32),
            pltpu.SemaphoreType.DMA,
        ],
        compiler_params=pltpu.CompilerParams(needs_layout_passes=False),
    )
    def gather_kernel(idx_hbm, table_hbm, out_hbm, idx_v, row_v, rows_v,
                      qt_v, gsem):
        wid = lax.axis_index("s") * _NUM_CORES + lax.axis_index("c")
        ubase = wid * units_per_w

        @pl.loop(0, units_per_w)
        def _unit(u):
            unit = ubase + u
            h = unit // chunks_per_h
            b0 = (unit % chunks_per_h) * _CB

            pltpu.sync_copy(idx_hbm.at[pl.ds(h * batch + b0, _CB)], idx_v)

            @pl.loop(0, _CB, step=_L)
            def _rows(j):
                v = idx_v[pl.ds(j, _L)]
                row_v[pl.ds(j, _L)] = lax.shift_right_logical(v, 1)

            pltpu.async_copy(table_hbm.at[row_v], rows_v, gsem).wait()

            @pl.loop(0, _CB, step=_L)
            def _tp(j):
                v = idx_v[pl.ds(j, _L)]
                half = lax.mul(lax.bitwise_and(v, 1), dim)
                jvec = lax.iota(jnp.int32, _L) + j
                for d in range(dim):
                    vals = plsc.load_gather(rows_v, [jvec, half + d])
                    qt_v[d, pl.ds(j, _L)] = vals

            pltpu.sync_copy(qt_v, out_hbm.at[h, :, pl.ds(b0, _CB)])

    return gather_kernel


def kernel(x, table):
    batch, hist = x.shape
    vocab, dim = table.shape
    idx_hm = jnp.transpose(x).reshape(-1).astype(jnp.int32)
    table_packed = table.reshape(vocab // 2, 2 * dim)
    q = _make_gather(batch, hist, vocab, dim)(idx_hm, table_packed)
    return jnp.transpose(q, (2, 0, 1))


# speedup vs baseline: 1.3486x; 1.3486x over previous
import functools

import jax
import jax.numpy as jnp
from jax import lax
from jax.experimental import pallas as pl
from jax.experimental.pallas import tpu as pltpu
from jax.experimental.pallas import tpu_sc as plsc

_NUM_CORES = 2
_NUM_SUBCORES = 16
_NUM_WORKERS = _NUM_CORES * _NUM_SUBCORES

_CB = 256
_L = 16


@functools.lru_cache(maxsize=None)
def _make_gather(batch: int, hist: int, vocab: int, dim: int):
    assert dim == 64 and vocab % 2 == 0 and batch % _CB == 0
    n_units = hist * (batch // _CB)
    assert n_units % _NUM_WORKERS == 0
    units_per_w = n_units // _NUM_WORKERS
    chunks_per_h = batch // _CB
    mesh = plsc.VectorSubcoreMesh(
        core_axis_name="c", subcore_axis_name="s",
        num_cores=_NUM_CORES, num_subcores=_NUM_SUBCORES)

    @functools.partial(
        pl.kernel,
        mesh=mesh,
        out_type=jax.ShapeDtypeStruct((hist, dim, batch), jnp.float32),
        scratch_types=[
            pltpu.VMEM((_CB,), jnp.int32),
            pltpu.VMEM((_CB,), jnp.int32),
            pltpu.VMEM((_CB, 2 * dim), jnp.float32),
            pltpu.VMEM((dim, _CB), jnp.float32),
            pltpu.SemaphoreType.DMA,
        ],
        compiler_params=pltpu.CompilerParams(needs_layout_passes=False),
    )
    def gather_kernel(idx_hbm, table_hbm, out_hbm, idx_v, row_v, rows_v,
                      qt_v, gsem):
        wid = lax.axis_index("s") * _NUM_CORES + lax.axis_index("c")
        ubase = wid * units_per_w

        @pl.loop(0, units_per_w)
        def _unit(u):
            unit = ubase + u
            h = unit // chunks_per_h
            b0 = (unit % chunks_per_h) * _CB

            pltpu.sync_copy(idx_hbm.at[pl.ds(h * batch + b0, _CB)], idx_v)

            @plsc.parallel_loop(0, _CB, step=_L, unroll=4)
            def _rows(j):
                v = idx_v[pl.ds(j, _L)]
                row_v[pl.ds(j, _L)] = lax.shift_right_logical(v, 1)

            pltpu.async_copy(table_hbm.at[row_v], rows_v, gsem).wait()

            @pl.loop(0, _CB, step=_L)
            def _tp(j):
                v = idx_v[pl.ds(j, _L)]
                half = lax.mul(lax.bitwise_and(v, 1), dim)
                jvec = lax.iota(jnp.int32, _L) + j

                @plsc.parallel_loop(0, dim, unroll=8)
                def _td(d):
                    vals = plsc.load_gather(rows_v, [jvec, half + d])
                    qt_v[d, pl.ds(j, _L)] = vals

            pltpu.sync_copy(qt_v, out_hbm.at[h, :, pl.ds(b0, _CB)])

    return gather_kernel


def kernel(x, table):
    batch, hist = x.shape
    vocab, dim = table.shape
    idx_hm = jnp.transpose(x).reshape(-1).astype(jnp.int32)
    table_packed = table.reshape(vocab // 2, 2 * dim)
    q = _make_gather(batch, hist, vocab, dim)(idx_hm, table_packed)
    return jnp.transpose(q, (2, 0, 1))
